# baseline (speedup 1.0000x reference)
import functools

import jax
import jax.numpy as jnp
import numpy as np
from jax import lax
from jax.experimental import pallas as pl
from jax.experimental.pallas import tpu as pltpu
from jax.experimental.pallas import tpu_sc as plsc

N_NODES = 10000
N_EDGES = 320000
CH = 128
AVG_NEIGH = 32.0

_NSUB = 16
_B = 64
_NB = 316
_EPT = _B * _NB
_EPAD = _NSUB * _EPT
_ROWS = 624
_TAIL = N_NODES - _NSUB * _ROWS


def _linup_body(nf_ref, w_ref, o_ref):
    o_ref[...] = jnp.dot(nf_ref[...], w_ref[...],
                         preferred_element_type=jnp.float32)


def _linear_up(node_feats, w_up_s):
    bn = 2000
    return pl.pallas_call(
        _linup_body,
        grid=(N_NODES // bn,),
        in_specs=[
            pl.BlockSpec((bn, CH), lambda i: (i, 0)),
            pl.BlockSpec((CH, CH), lambda i: (0, 0)),
        ],
        out_specs=pl.BlockSpec((bn, CH), lambda i: (i, 0)),
        out_shape=jax.ShapeDtypeStruct((N_NODES, CH), jnp.float32),
    )(node_feats, w_up_s)


def _edgew_body(ef_ref, ea_ref, w1_ref, w2_ref, w3_ref, w4_ref, wv_ref):
    h = jax.nn.silu(jnp.dot(ef_ref[...], w1_ref[...],
                            preferred_element_type=jnp.float32))
    h = jax.nn.silu(jnp.dot(h, w2_ref[...],
                            preferred_element_type=jnp.float32))
    h = jax.nn.silu(jnp.dot(h, w3_ref[...],
                            preferred_element_type=jnp.float32))
    tpw = jnp.dot(h, w4_ref[...], preferred_element_type=jnp.float32)
    ea = ea_ref[...]
    w0 = tpw[:, :CH]
    w1t = tpw[:, CH:]
    wv_ref[0] = (w0 * ea[:, 0:1]).astype(jnp.bfloat16)
    wv_ref[1] = (w1t * ea[:, 1:2]).astype(jnp.bfloat16)
    wv_ref[2] = (w1t * ea[:, 2:3]).astype(jnp.bfloat16)
    wv_ref[3] = (w1t * ea[:, 3:4]).astype(jnp.bfloat16)


def _edge_weights(edge_feats, edge_attrs, w1s, w2s, w3s, w4s):
    be = 2528
    return pl.pallas_call(
        _edgew_body,
        grid=(_EPAD // be,),
        in_specs=[
            pl.BlockSpec((be, 8), lambda i: (i, 0)),
            pl.BlockSpec((be, 4), lambda i: (i, 0)),
            pl.BlockSpec((8, 64), lambda i: (0, 0)),
            pl.BlockSpec((64, 64), lambda i: (0, 0)),
            pl.BlockSpec((64, 64), lambda i: (0, 0)),
            pl.BlockSpec((64, 2 * CH), lambda i: (0, 0)),
        ],
        out_specs=pl.BlockSpec((4, be, CH), lambda i: (0, i, 0)),
        out_shape=jax.ShapeDtypeStruct((4, _EPAD, CH), jnp.bfloat16),
    )(edge_feats, edge_attrs, w1s, w2s, w3s, w4s)


def _sc_body(snd_hbm, rcv_hbm, x_hbm, wv_hbm, out_hbm,
             snd0, snd1, rcv0, rcv1, ridx0, ridx1,
             xs0, xs1, wv0, wv1, prod0, prod1, acc_sh,
             is0, is1, ir0, ir1, gs0, gs1, ws0, ws1, ss0, ss1):
    cid = lax.axis_index("c")
    sid = lax.axis_index("s")
    ebase = sid * _EPT

    snd_b = (snd0, snd1)
    rcv_b = (rcv0, rcv1)
    ridx_b = (ridx0, ridx1)
    xs_b = (xs0, xs1)
    wv_b = (wv0, wv1)
    prod_b = (prod0, prod1)
    is_b = (is0, is1)
    ir_b = (ir0, ir1)
    gs_b = (gs0, gs1)
    ws_b = (ws0, ws1)
    ss_b = (ss0, ss1)

    for r in range(2):
        chunk = r * 2 + cid
        wvbase = chunk * (_EPAD // 2) + sid * (_EPT // 2)

        def zrow(i, carry):
            for k in range(CH // 16):
                prod0[i, pl.ds(k * 16, 16)] = jnp.zeros((16,), jnp.float32)
            return carry
        lax.fori_loop(0, _B, zrow, 0)
        for z in range(9):
            pltpu.sync_copy(prod0,
                            acc_sh.at[pl.ds(sid * _ROWS + z * _B, _B)])
        pltpu.sync_copy(prod0.at[pl.ds(0, 48)],
                        acc_sh.at[pl.ds(sid * _ROWS + 9 * _B, 48)])

        @pl.when(sid == 0)
        def _zero_tail():
            pltpu.sync_copy(prod0.at[pl.ds(0, _TAIL)],
                            acc_sh.at[pl.ds(_NSUB * _ROWS, _TAIL)])
        plsc.subcore_barrier()

        def idx_start(i, b):
            eb = ebase + i * _B
            pltpu.async_copy(snd_hbm.at[pl.ds(eb, _B)], snd_b[b], is_b[b])
            pltpu.async_copy(rcv_hbm.at[pl.ds(eb, _B)], rcv_b[b], ir_b[b])

        def idx_wait(i, b):
            eb = ebase + i * _B
            pltpu.make_async_copy(snd_hbm.at[pl.ds(eb, _B)], snd_b[b],
                                  is_b[b]).wait()
            pltpu.make_async_copy(rcv_hbm.at[pl.ds(eb, _B)], rcv_b[b],
                                  ir_b[b]).wait()

        def fetch_start(i, b):
            pltpu.async_copy(x_hbm.at[snd_b[b]], xs_b[b], gs_b[b])
            pltpu.async_copy(wv_hbm.at[pl.ds(wvbase + i * (_B // 2),
                                             _B // 2)],
                             wv_b[b], ws_b[b])

        def fetch_wait(i, b):
            pltpu.make_async_copy(x_hbm.at[snd_b[b]], xs_b[b],
                                  gs_b[b]).wait()
            pltpu.make_async_copy(wv_hbm.at[pl.ds(wvbase + i * (_B // 2),
                                                  _B // 2)],
                                  wv_b[b], ws_b[b]).wait()

        def scatter_wait(b):
            pltpu.make_async_copy(prod_b[b], acc_sh.at[ridx_b[b]],
                                  ss_b[b]).wait()

        idx_start(0, 0)
        idx_start(1, 1)
        idx_wait(0, 0)
        fetch_start(0, 0)

        def half(i, b):
            q = 1 - b
            xs_p, wv_p, prod_p = xs_b[b], wv_b[b], prod_b[b]

            @pl.when(i >= 2)
            def _():
                scatter_wait(b)
            fetch_wait(i, b)

            for t in range(_B // 16):
                ridx_b[b][pl.ds(t * 16, 16)] = rcv_b[b][pl.ds(t * 16, 16)]

            @pl.when(i + 2 < _NB)
            def _():
                idx_start(i + 2, b)

            @pl.when(i + 1 < _NB)
            def _():
                idx_wait(i + 1, q)
                fetch_start(i + 1, q)

            @plsc.parallel_loop(0, _B // 2, 1, unroll=2)
            def _mul(m):
                for h in range(2):
                    e = 2 * m + h
                    for t in range(CH // 32):
                        wvv = wv_p[m, pl.ds(h * 64 + t * 16, 16)]
                        wa = lax.bitcast_convert_type(wvv << 16,
                                                      jnp.float32)
                        wb = lax.bitcast_convert_type(
                            wvv & jnp.int32(-65536), jnp.float32)
                        prod_p[e, pl.ds(t * 32, 16)] = (
                            xs_p[e, pl.ds(t * 32, 16)] * wa)
                        prod_p[e, pl.ds(t * 32 + 16, 16)] = (
                            xs_p[e, pl.ds(t * 32 + 16, 16)] * wb)

            pltpu.async_copy(prod_p, acc_sh.at[ridx_b[b]], ss_b[b],
                             add=True)

        def pair(j, carry):
            half(2 * j, 0)
            half(2 * j + 1, 1)
            return carry
        lax.fori_loop(0, _NB // 2, pair, 0)

        scatter_wait(0)
        scatter_wait(1)
        plsc.subcore_barrier()

        pltpu.sync_copy(acc_sh.at[pl.ds(sid * _ROWS, _ROWS)],
                        out_hbm.at[pl.ds(chunk * N_NODES + sid * _ROWS,
                                         _ROWS)])

        @pl.when(sid == 0)
        def _dump_tail():
            pltpu.sync_copy(
                acc_sh.at[pl.ds(_NSUB * _ROWS, _TAIL)],
                out_hbm.at[pl.ds(chunk * N_NODES + _NSUB * _ROWS, _TAIL)])


def _sc_message(snd, rcv, x, wv2d):
    mesh = plsc.VectorSubcoreMesh(core_axis_name="c", subcore_axis_name="s")
    dma = pltpu.SemaphoreType.DMA
    k = functools.partial(
        pl.kernel,
        mesh=mesh,
        out_type=jax.ShapeDtypeStruct((4 * N_NODES, CH), jnp.float32),
        scratch_types=[
            pltpu.VMEM((_B,), jnp.int32),
            pltpu.VMEM((_B,), jnp.int32),
            pltpu.VMEM((_B,), jnp.int32),
            pltpu.VMEM((_B,), jnp.int32),
            pltpu.VMEM((_B,), jnp.int32),
            pltpu.VMEM((_B,), jnp.int32),
            pltpu.VMEM((_B, CH), jnp.float32),
            pltpu.VMEM((_B, CH), jnp.float32),
            pltpu.VMEM((_B // 2, CH), jnp.int32),
            pltpu.VMEM((_B // 2, CH), jnp.int32),
            pltpu.VMEM((_B, CH), jnp.float32),
            pltpu.VMEM((_B, CH), jnp.float32),
            pltpu.VMEM_SHARED((N_NODES, CH), jnp.float32),
            dma, dma, dma, dma, dma, dma, dma, dma, dma, dma,
        ],
    )(_sc_body)
    return k(snd, rcv, x, wv2d)


def _outlin_body(m_ref, w_ref, o_ref):
    o_ref[0] = jnp.dot(m_ref[0], w_ref[0],
                       preferred_element_type=jnp.float32)


def _out_linear(msg, w_stack):
    bn = 2000
    return pl.pallas_call(
        _outlin_body,
        grid=(4, N_NODES // bn),
        in_specs=[
            pl.BlockSpec((1, bn, CH), lambda c, i: (c, i, 0)),
            pl.BlockSpec((1, CH, CH), lambda c, i: (c, 0, 0)),
        ],
        out_specs=pl.BlockSpec((1, bn, CH), lambda c, i: (c, i, 0)),
        out_shape=jax.ShapeDtypeStruct((4, N_NODES, CH), jnp.float32),
    )(msg, w_stack)


def kernel(node_attrs, node_feats, edge_attrs, edge_feats, edge_index,
           W_up, W1, W2, W3, W4, W_lin0, W_lin1):
    del node_attrs
    pad = _EPAD - N_EDGES
    snd = jnp.pad(edge_index[0], (0, pad))
    rcv = jnp.pad(edge_index[1], (0, pad))
    ef = jnp.pad(edge_feats, ((0, pad), (0, 0)))
    ea = jnp.pad(edge_attrs, ((0, pad), (0, 0)))
    w_up_s = W_up * np.float32(1.0 / np.sqrt(CH))
    w1s = W1 * np.float32(1.0 / np.sqrt(8.0))
    w2s = W2 * np.float32(1.0 / np.sqrt(64.0))
    w3s = W3 * np.float32(1.0 / np.sqrt(64.0))
    w4s = W4 * np.float32(1.0 / np.sqrt(64.0))
    out_scale = np.float32(1.0 / (np.sqrt(CH) * AVG_NEIGH))
    w_stack = jnp.stack([W_lin0, W_lin1, W_lin1, W_lin1], axis=0) * out_scale
    tj = np.arange(128)
    t, j = (tj // 2) // 16, (tj // 2) % 16
    sigma = 32 * t + 16 * (tj % 2) + j
    w4s = jnp.concatenate([w4s[:, :CH][:, sigma], w4s[:, CH:][:, sigma]],
                          axis=1)

    x = _linear_up(node_feats, w_up_s)
    wv = _edge_weights(ef, ea, w1s, w2s, w3s, w4s)
    wv_i = lax.bitcast_convert_type(
        wv.reshape(4 * _EPAD // 2, CH, 2), jnp.int32)
    msg2d = _sc_message(snd, rcv, x, wv_i)
    msg = msg2d.reshape(4, N_NODES, CH)
    m = _out_linear(msg, w_stack)
    m1 = jnp.stack([m[1], m[2], m[3]], axis=-1).reshape(N_NODES, 3 * CH)
    return jnp.concatenate([m[0], m1], axis=1)


# speedup vs baseline: 26.2647x; 26.2647x over previous
import functools

import jax
import jax.numpy as jnp
import numpy as np
from jax import lax
from jax.experimental import pallas as pl
from jax.experimental.pallas import tpu as pltpu
from jax.experimental.pallas import tpu_sc as plsc

N_NODES = 10000
N_EDGES = 320000
CH = 128
AVG_NEIGH = 32.0

_NSUB = 16
_B = 64
_NB = 316
_EPT = _B * _NB
_EPAD = _NSUB * _EPT
_ROWS = 624
_TAIL = N_NODES - _NSUB * _ROWS


def _linup_body(nf_ref, w_ref, o_ref):
    o_ref[...] = jnp.dot(nf_ref[...], w_ref[...],
                         preferred_element_type=jnp.float32)


def _linear_up(node_feats, w_up_s):
    bn = 2000
    return pl.pallas_call(
        _linup_body,
        grid=(N_NODES // bn,),
        in_specs=[
            pl.BlockSpec((bn, CH), lambda i: (i, 0)),
            pl.BlockSpec((CH, CH), lambda i: (0, 0)),
        ],
        out_specs=pl.BlockSpec((bn, CH), lambda i: (i, 0)),
        out_shape=jax.ShapeDtypeStruct((N_NODES, CH), jnp.float32),
    )(node_feats, w_up_s)


def _mlp_tc(x, w1_ref, w2_ref, w3_ref, w4_ref):
    h = jax.nn.silu(jnp.dot(x, w1_ref[...],
                            preferred_element_type=jnp.float32))
    h = jax.nn.silu(jnp.dot(h, w2_ref[...],
                            preferred_element_type=jnp.float32))
    h = jax.nn.silu(jnp.dot(h, w3_ref[...],
                            preferred_element_type=jnp.float32))
    return jnp.dot(h, w4_ref[...], preferred_element_type=jnp.float32)


def _rne_bf16_bits(a):
    u = jax.lax.bitcast_convert_type(a, jnp.int32)
    return jax.lax.shift_right_logical(
        u + jnp.int32(0x7FFF) + (jax.lax.shift_right_logical(u, 16)
                                 & jnp.int32(1)), 16)


def _edgew_body(ef_ref, ea_ref, w1_ref, w2_ref, w3_ref, w4_ref, wv_ref):
    efp = ef_ref[...]
    eap = ea_ref[...]
    tpe = _mlp_tc(efp[:, :8], w1_ref, w2_ref, w3_ref, w4_ref)
    tpo = _mlp_tc(efp[:, 8:], w1_ref, w2_ref, w3_ref, w4_ref)
    w0e, w1e = tpe[:, :CH], tpe[:, CH:]
    w0o, w1o = tpo[:, :CH], tpo[:, CH:]

    def pack(a, b):
        return _rne_bf16_bits(a) | (_rne_bf16_bits(b) << 16)

    wv_ref[0] = pack(w0e * eap[:, 0:1], w0o * eap[:, 4:5])
    wv_ref[1] = pack(w1e * eap[:, 1:2], w1o * eap[:, 5:6])
    wv_ref[2] = pack(w1e * eap[:, 2:3], w1o * eap[:, 6:7])
    wv_ref[3] = pack(w1e * eap[:, 3:4], w1o * eap[:, 7:8])


def _edge_weights(efp, eap, w1s, w2s, w3s, w4s):
    be2 = 2048
    return pl.pallas_call(
        _edgew_body,
        grid=(_EPAD // 2 // be2,),
        in_specs=[
            pl.BlockSpec((be2, 16), lambda i: (i, 0)),
            pl.BlockSpec((be2, 8), lambda i: (i, 0)),
            pl.BlockSpec((8, 64), lambda i: (0, 0)),
            pl.BlockSpec((64, 64), lambda i: (0, 0)),
            pl.BlockSpec((64, 64), lambda i: (0, 0)),
            pl.BlockSpec((64, 2 * CH), lambda i: (0, 0)),
        ],
        out_specs=pl.BlockSpec((4, be2, CH), lambda i: (0, i, 0)),
        out_shape=jax.ShapeDtypeStruct((4, _EPAD // 2, CH), jnp.int32),
    )(efp, eap, w1s, w2s, w3s, w4s)


def _sc_body(snd_hbm, rcv_hbm, x_hbm, wv_hbm, out_hbm,
             snd0, snd1, rcv0, rcv1, ridx0, ridx1,
             xs0, xs1, wv0, wv1, prod0, prod1, acc_sh,
             is0, is1, ir0, ir1, gs0, gs1, ws0, ws1, ss0, ss1):
    cid = lax.axis_index("c")
    sid = lax.axis_index("s")
    ebase = sid * _EPT

    snd_b = (snd0, snd1)
    rcv_b = (rcv0, rcv1)
    ridx_b = (ridx0, ridx1)
    xs_b = (xs0, xs1)
    wv_b = (wv0, wv1)
    prod_b = (prod0, prod1)
    is_b = (is0, is1)
    ir_b = (ir0, ir1)
    gs_b = (gs0, gs1)
    ws_b = (ws0, ws1)
    ss_b = (ss0, ss1)

    for r in range(2):
        chunk = r * 2 + cid
        wvbase = chunk * (_EPAD // 2) + sid * (_EPT // 2)

        def zrow(i, carry):
            for k in range(CH // 16):
                prod0[i, pl.ds(k * 16, 16)] = jnp.zeros((16,), jnp.float32)
            return carry
        lax.fori_loop(0, _B, zrow, 0)
        for z in range(9):
            pltpu.sync_copy(prod0,
                            acc_sh.at[pl.ds(sid * _ROWS + z * _B, _B)])
        pltpu.sync_copy(prod0.at[pl.ds(0, 48)],
                        acc_sh.at[pl.ds(sid * _ROWS + 9 * _B, 48)])

        @pl.when(sid == 0)
        def _zero_tail():
            pltpu.sync_copy(prod0.at[pl.ds(0, _TAIL)],
                            acc_sh.at[pl.ds(_NSUB * _ROWS, _TAIL)])
        plsc.subcore_barrier()

        def idx_start(i, b):
            eb = ebase + i * _B
            pltpu.async_copy(snd_hbm.at[pl.ds(eb, _B)], snd_b[b], is_b[b])
            pltpu.async_copy(rcv_hbm.at[pl.ds(eb, _B)], rcv_b[b], ir_b[b])

        def idx_wait(i, b):
            eb = ebase + i * _B
            pltpu.make_async_copy(snd_hbm.at[pl.ds(eb, _B)], snd_b[b],
                                  is_b[b]).wait()
            pltpu.make_async_copy(rcv_hbm.at[pl.ds(eb, _B)], rcv_b[b],
                                  ir_b[b]).wait()

        def fetch_start(i, b):
            pltpu.async_copy(x_hbm.at[snd_b[b]], xs_b[b], gs_b[b])
            pltpu.async_copy(wv_hbm.at[pl.ds(wvbase + i * (_B // 2),
                                             _B // 2)],
                             wv_b[b], ws_b[b])

        def fetch_wait(i, b):
            pltpu.make_async_copy(x_hbm.at[snd_b[b]], xs_b[b],
                                  gs_b[b]).wait()
            pltpu.make_async_copy(wv_hbm.at[pl.ds(wvbase + i * (_B // 2),
                                                  _B // 2)],
                                  wv_b[b], ws_b[b]).wait()

        def scatter_wait(b):
            pltpu.make_async_copy(prod_b[b], acc_sh.at[ridx_b[b]],
                                  ss_b[b]).wait()

        idx_start(0, 0)
        idx_start(1, 1)
        idx_wait(0, 0)
        fetch_start(0, 0)

        def half(i, b):
            q = 1 - b
            xs_p, wv_p, prod_p = xs_b[b], wv_b[b], prod_b[b]

            @pl.when(i >= 2)
            def _():
                scatter_wait(b)
            fetch_wait(i, b)

            for t in range(_B // 16):
                ridx_b[b][pl.ds(t * 16, 16)] = rcv_b[b][pl.ds(t * 16, 16)]

            @pl.when(i + 2 < _NB)
            def _():
                idx_start(i + 2, b)

            @pl.when(i + 1 < _NB)
            def _():
                idx_wait(i + 1, q)
                fetch_start(i + 1, q)

            @plsc.parallel_loop(0, _B // 2, 1, unroll=2)
            def _mul(m):
                e0 = 2 * m
                for t in range(CH // 16):
                    wvv = wv_p[m, pl.ds(t * 16, 16)]
                    wa = lax.bitcast_convert_type(wvv << 16, jnp.float32)
                    wb = lax.bitcast_convert_type(
                        wvv & jnp.int32(-65536), jnp.float32)
                    prod_p[e0, pl.ds(t * 16, 16)] = (
                        xs_p[e0, pl.ds(t * 16, 16)] * wa)
                    prod_p[e0 + 1, pl.ds(t * 16, 16)] = (
                        xs_p[e0 + 1, pl.ds(t * 16, 16)] * wb)

            pltpu.async_copy(prod_p, acc_sh.at[ridx_b[b]], ss_b[b],
                             add=True)

        def pair(j, carry):
            half(2 * j, 0)
            half(2 * j + 1, 1)
            return carry
        lax.fori_loop(0, _NB // 2, pair, 0)

        scatter_wait(0)
        scatter_wait(1)
        plsc.subcore_barrier()

        pltpu.sync_copy(acc_sh.at[pl.ds(sid * _ROWS, _ROWS)],
                        out_hbm.at[pl.ds(chunk * N_NODES + sid * _ROWS,
                                         _ROWS)])

        @pl.when(sid == 0)
        def _dump_tail():
            pltpu.sync_copy(
                acc_sh.at[pl.ds(_NSUB * _ROWS, _TAIL)],
                out_hbm.at[pl.ds(chunk * N_NODES + _NSUB * _ROWS, _TAIL)])


def _sc_message(snd, rcv, x, wv2d):
    mesh = plsc.VectorSubcoreMesh(core_axis_name="c", subcore_axis_name="s")
    dma = pltpu.SemaphoreType.DMA
    k = functools.partial(
        pl.kernel,
        mesh=mesh,
        out_type=jax.ShapeDtypeStruct((4 * N_NODES, CH), jnp.float32),
        scratch_types=[
            pltpu.VMEM((_B,), jnp.int32),
            pltpu.VMEM((_B,), jnp.int32),
            pltpu.VMEM((_B,), jnp.int32),
            pltpu.VMEM((_B,), jnp.int32),
            pltpu.VMEM((_B,), jnp.int32),
            pltpu.VMEM((_B,), jnp.int32),
            pltpu.VMEM((_B, CH), jnp.float32),
            pltpu.VMEM((_B, CH), jnp.float32),
            pltpu.VMEM((_B // 2, CH), jnp.int32),
            pltpu.VMEM((_B // 2, CH), jnp.int32),
            pltpu.VMEM((_B, CH), jnp.float32),
            pltpu.VMEM((_B, CH), jnp.float32),
            pltpu.VMEM_SHARED((N_NODES, CH), jnp.float32),
            dma, dma, dma, dma, dma, dma, dma, dma, dma, dma,
        ],
    )(_sc_body)
    return k(snd, rcv, x, wv2d)


def _outlin_body(m_ref, w_ref, o_ref):
    o_ref[0] = jnp.dot(m_ref[0], w_ref[0],
                       preferred_element_type=jnp.float32)


def _out_linear(msg, w_stack):
    bn = 2000
    return pl.pallas_call(
        _outlin_body,
        grid=(4, N_NODES // bn),
        in_specs=[
            pl.BlockSpec((1, bn, CH), lambda c, i: (c, i, 0)),
            pl.BlockSpec((1, CH, CH), lambda c, i: (c, 0, 0)),
        ],
        out_specs=pl.BlockSpec((1, bn, CH), lambda c, i: (c, i, 0)),
        out_shape=jax.ShapeDtypeStruct((4, N_NODES, CH), jnp.float32),
    )(msg, w_stack)


def kernel(node_attrs, node_feats, edge_attrs, edge_feats, edge_index,
           W_up, W1, W2, W3, W4, W_lin0, W_lin1):
    del node_attrs
    pad = _EPAD - N_EDGES
    snd = jnp.pad(edge_index[0], (0, pad))
    rcv = jnp.pad(edge_index[1], (0, pad))
    ef = jnp.pad(edge_feats, ((0, pad), (0, 0)))
    ea = jnp.pad(edge_attrs, ((0, pad), (0, 0)))
    w_up_s = W_up * np.float32(1.0 / np.sqrt(CH))
    w1s = W1 * np.float32(1.0 / np.sqrt(8.0))
    w2s = W2 * np.float32(1.0 / np.sqrt(64.0))
    w3s = W3 * np.float32(1.0 / np.sqrt(64.0))
    w4s = W4 * np.float32(1.0 / np.sqrt(64.0))
    out_scale = np.float32(1.0 / (np.sqrt(CH) * AVG_NEIGH))
    w_stack = jnp.stack([W_lin0, W_lin1, W_lin1, W_lin1], axis=0) * out_scale

    x = _linear_up(node_feats, w_up_s)
    wv = _edge_weights(ef.reshape(_EPAD // 2, 16), ea.reshape(_EPAD // 2, 8),
                       w1s, w2s, w3s, w4s)
    msg2d = _sc_message(snd, rcv, x, wv.reshape(4 * _EPAD // 2, CH))
    msg = msg2d.reshape(4, N_NODES, CH)
    m = _out_linear(msg, w_stack)
    m1 = jnp.stack([m[1], m[2], m[3]], axis=-1).reshape(N_NODES, 3 * CH)
    return jnp.concatenate([m[0], m1], axis=1)


# speedup vs baseline: 26.3165x; 1.0020x over previous
import functools

import jax
import jax.numpy as jnp
import numpy as np
from jax import lax
from jax.experimental import pallas as pl
from jax.experimental.pallas import tpu as pltpu
from jax.experimental.pallas import tpu_sc as plsc

N_NODES = 10000
N_EDGES = 320000
CH = 128
AVG_NEIGH = 32.0

_NSUB = 16
_B = 64
_NB = 316
_EPT = _B * _NB
_EPAD = _NSUB * _EPT
_ROWS = 624
_TAIL = N_NODES - _NSUB * _ROWS


def _linup_body(nf_ref, w_ref, o_ref):
    o_ref[...] = jnp.dot(nf_ref[...], w_ref[...],
                         preferred_element_type=jnp.float32)


def _linear_up(node_feats, w_up_s):
    bn = 2000
    return pl.pallas_call(
        _linup_body,
        grid=(N_NODES // bn,),
        in_specs=[
            pl.BlockSpec((bn, CH), lambda i: (i, 0)),
            pl.BlockSpec((CH, CH), lambda i: (0, 0)),
        ],
        out_specs=pl.BlockSpec((bn, CH), lambda i: (i, 0)),
        out_shape=jax.ShapeDtypeStruct((N_NODES, CH), jnp.float32),
    )(node_feats, w_up_s)


def _mlp_tc(x, w1_ref, w2_ref, w3_ref, w4_ref):
    h = jax.nn.silu(jnp.dot(x.astype(jnp.bfloat16), w1_ref[...],
                            preferred_element_type=jnp.float32))
    h = jax.nn.silu(jnp.dot(h.astype(jnp.bfloat16), w2_ref[...],
                            preferred_element_type=jnp.float32))
    h = jax.nn.silu(jnp.dot(h.astype(jnp.bfloat16), w3_ref[...],
                            preferred_element_type=jnp.float32))
    return jnp.dot(h.astype(jnp.bfloat16), w4_ref[...],
                   preferred_element_type=jnp.float32)


def _rne_bf16_bits(a):
    u = jax.lax.bitcast_convert_type(a, jnp.int32)
    return jax.lax.shift_right_logical(
        u + jnp.int32(0x7FFF) + (jax.lax.shift_right_logical(u, 16)
                                 & jnp.int32(1)), 16)


def _edgew_body(ef_ref, ea_ref, w1_ref, w2_ref, w3_ref, w4_ref, wv_ref):
    efp = ef_ref[...]
    eap = ea_ref[...]
    tpe = _mlp_tc(efp[:, :8], w1_ref, w2_ref, w3_ref, w4_ref)
    tpo = _mlp_tc(efp[:, 8:], w1_ref, w2_ref, w3_ref, w4_ref)
    w0e, w1e = tpe[:, :CH], tpe[:, CH:]
    w0o, w1o = tpo[:, :CH], tpo[:, CH:]

    def pack(a, b):
        return _rne_bf16_bits(a) | (_rne_bf16_bits(b) << 16)

    wv_ref[0] = pack(w0e * eap[:, 0:1], w0o * eap[:, 4:5])
    wv_ref[1] = pack(w1e * eap[:, 1:2], w1o * eap[:, 5:6])
    wv_ref[2] = pack(w1e * eap[:, 2:3], w1o * eap[:, 6:7])
    wv_ref[3] = pack(w1e * eap[:, 3:4], w1o * eap[:, 7:8])


def _edge_weights(efp, eap, w1s, w2s, w3s, w4s):
    be2 = 2048
    return pl.pallas_call(
        _edgew_body,
        grid=(_EPAD // 2 // be2,),
        in_specs=[
            pl.BlockSpec((be2, 16), lambda i: (i, 0)),
            pl.BlockSpec((be2, 8), lambda i: (i, 0)),
            pl.BlockSpec((8, 64), lambda i: (0, 0)),
            pl.BlockSpec((64, 64), lambda i: (0, 0)),
            pl.BlockSpec((64, 64), lambda i: (0, 0)),
            pl.BlockSpec((64, 2 * CH), lambda i: (0, 0)),
        ],
        out_specs=pl.BlockSpec((4, be2, CH), lambda i: (0, i, 0)),
        out_shape=jax.ShapeDtypeStruct((4, _EPAD // 2, CH), jnp.int32),
    )(efp, eap, w1s, w2s, w3s, w4s)


def _sc_body(snd_hbm, rcv_hbm, x_hbm, wv_hbm, out_hbm,
             snd0, snd1, rcv0, rcv1, ridx0, ridx1,
             xs0, xs1, wv0, wv1, prod0, prod1, acc_sh,
             is0, is1, ir0, ir1, gs0, gs1, ws0, ws1, ss0, ss1):
    cid = lax.axis_index("c")
    sid = lax.axis_index("s")
    ebase = sid * _EPT

    snd_b = (snd0, snd1)
    rcv_b = (rcv0, rcv1)
    ridx_b = (ridx0, ridx1)
    xs_b = (xs0, xs1)
    wv_b = (wv0, wv1)
    prod_b = (prod0, prod1)
    is_b = (is0, is1)
    ir_b = (ir0, ir1)
    gs_b = (gs0, gs1)
    ws_b = (ws0, ws1)
    ss_b = (ss0, ss1)

    for r in range(2):
        chunk = r * 2 + cid
        wvbase = chunk * (_EPAD // 2) + sid * (_EPT // 2)

        def zrow(i, carry):
            for k in range(CH // 16):
                prod0[i, pl.ds(k * 16, 16)] = jnp.zeros((16,), jnp.float32)
            return carry
        lax.fori_loop(0, _B, zrow, 0)
        for z in range(9):
            pltpu.sync_copy(prod0,
                            acc_sh.at[pl.ds(sid * _ROWS + z * _B, _B)])
        pltpu.sync_copy(prod0.at[pl.ds(0, 48)],
                        acc_sh.at[pl.ds(sid * _ROWS + 9 * _B, 48)])

        @pl.when(sid == 0)
        def _zero_tail():
            pltpu.sync_copy(prod0.at[pl.ds(0, _TAIL)],
                            acc_sh.at[pl.ds(_NSUB * _ROWS, _TAIL)])
        plsc.subcore_barrier()

        def idx_start(i, b):
            eb = ebase + i * _B
            pltpu.async_copy(snd_hbm.at[pl.ds(eb, _B)], snd_b[b], is_b[b])
            pltpu.async_copy(rcv_hbm.at[pl.ds(eb, _B)], rcv_b[b], ir_b[b])

        def idx_wait(i, b):
            eb = ebase + i * _B
            pltpu.make_async_copy(snd_hbm.at[pl.ds(eb, _B)], snd_b[b],
                                  is_b[b]).wait()
            pltpu.make_async_copy(rcv_hbm.at[pl.ds(eb, _B)], rcv_b[b],
                                  ir_b[b]).wait()

        def fetch_start(i, b):
            pltpu.async_copy(x_hbm.at[snd_b[b]], xs_b[b], gs_b[b])
            pltpu.async_copy(wv_hbm.at[pl.ds(wvbase + i * (_B // 2),
                                             _B // 2)],
                             wv_b[b], ws_b[b])

        def fetch_wait(i, b):
            pltpu.make_async_copy(x_hbm.at[snd_b[b]], xs_b[b],
                                  gs_b[b]).wait()
            pltpu.make_async_copy(wv_hbm.at[pl.ds(wvbase + i * (_B // 2),
                                                  _B // 2)],
                                  wv_b[b], ws_b[b]).wait()

        def scatter_wait(b):
            pltpu.make_async_copy(prod_b[b], acc_sh.at[ridx_b[b]],
                                  ss_b[b]).wait()

        idx_start(0, 0)
        idx_start(1, 1)
        idx_wait(0, 0)
        fetch_start(0, 0)

        def half(i, b):
            q = 1 - b
            xs_p, wv_p, prod_p = xs_b[b], wv_b[b], prod_b[b]

            @pl.when(i >= 2)
            def _():
                scatter_wait(b)
            fetch_wait(i, b)

            for t in range(_B // 16):
                ridx_b[b][pl.ds(t * 16, 16)] = rcv_b[b][pl.ds(t * 16, 16)]

            @pl.when(i + 2 < _NB)
            def _():
                idx_start(i + 2, b)

            @pl.when(i + 1 < _NB)
            def _():
                idx_wait(i + 1, q)
                fetch_start(i + 1, q)

            @plsc.parallel_loop(0, _B // 2, 1, unroll=4)
            def _mul(m):
                e0 = 2 * m
                for t in range(CH // 16):
                    wvv = wv_p[m, pl.ds(t * 16, 16)]
                    wa = lax.bitcast_convert_type(wvv << 16, jnp.float32)
                    wb = lax.bitcast_convert_type(
                        wvv & jnp.int32(-65536), jnp.float32)
                    prod_p[e0, pl.ds(t * 16, 16)] = (
                        xs_p[e0, pl.ds(t * 16, 16)] * wa)
                    prod_p[e0 + 1, pl.ds(t * 16, 16)] = (
                        xs_p[e0 + 1, pl.ds(t * 16, 16)] * wb)

            pltpu.async_copy(prod_p, acc_sh.at[ridx_b[b]], ss_b[b],
                             add=True)

        def pair(j, carry):
            half(2 * j, 0)
            half(2 * j + 1, 1)
            return carry
        lax.fori_loop(0, _NB // 2, pair, 0)

        scatter_wait(0)
        scatter_wait(1)
        plsc.subcore_barrier()

        pltpu.sync_copy(acc_sh.at[pl.ds(sid * _ROWS, _ROWS)],
                        out_hbm.at[pl.ds(chunk * N_NODES + sid * _ROWS,
                                         _ROWS)])

        @pl.when(sid == 0)
        def _dump_tail():
            pltpu.sync_copy(
                acc_sh.at[pl.ds(_NSUB * _ROWS, _TAIL)],
                out_hbm.at[pl.ds(chunk * N_NODES + _NSUB * _ROWS, _TAIL)])


def _sc_message(snd, rcv, x, wv2d):
    mesh = plsc.VectorSubcoreMesh(core_axis_name="c", subcore_axis_name="s")
    dma = pltpu.SemaphoreType.DMA
    k = functools.partial(
        pl.kernel,
        mesh=mesh,
        out_type=jax.ShapeDtypeStruct((4 * N_NODES, CH), jnp.float32),
        scratch_types=[
            pltpu.VMEM((_B,), jnp.int32),
            pltpu.VMEM((_B,), jnp.int32),
            pltpu.VMEM((_B,), jnp.int32),
            pltpu.VMEM((_B,), jnp.int32),
            pltpu.VMEM((_B,), jnp.int32),
            pltpu.VMEM((_B,), jnp.int32),
            pltpu.VMEM((_B, CH), jnp.float32),
            pltpu.VMEM((_B, CH), jnp.float32),
            pltpu.VMEM((_B // 2, CH), jnp.int32),
            pltpu.VMEM((_B // 2, CH), jnp.int32),
            pltpu.VMEM((_B, CH), jnp.float32),
            pltpu.VMEM((_B, CH), jnp.float32),
            pltpu.VMEM_SHARED((N_NODES, CH), jnp.float32),
            dma, dma, dma, dma, dma, dma, dma, dma, dma, dma,
        ],
    )(_sc_body)
    return k(snd, rcv, x, wv2d)


def _outlin_body(m_ref, w_ref, o_ref):
    o_ref[0] = jnp.dot(m_ref[0], w_ref[0],
                       preferred_element_type=jnp.float32)


def _out_linear(msg, w_stack):
    bn = 2000
    return pl.pallas_call(
        _outlin_body,
        grid=(4, N_NODES // bn),
        in_specs=[
            pl.BlockSpec((1, bn, CH), lambda c, i: (c, i, 0)),
            pl.BlockSpec((1, CH, CH), lambda c, i: (c, 0, 0)),
        ],
        out_specs=pl.BlockSpec((1, bn, CH), lambda c, i: (c, i, 0)),
        out_shape=jax.ShapeDtypeStruct((4, N_NODES, CH), jnp.float32),
    )(msg, w_stack)


def kernel(node_attrs, node_feats, edge_attrs, edge_feats, edge_index,
           W_up, W1, W2, W3, W4, W_lin0, W_lin1):
    del node_attrs
    pad = _EPAD - N_EDGES
    snd = jnp.pad(edge_index[0], (0, pad))
    rcv = jnp.pad(edge_index[1], (0, pad))
    ef = jnp.pad(edge_feats, ((0, pad), (0, 0)))
    ea = jnp.pad(edge_attrs, ((0, pad), (0, 0)))
    w_up_s = W_up * np.float32(1.0 / np.sqrt(CH))
    w1s = (W1 * np.float32(1.0 / np.sqrt(8.0))).astype(jnp.bfloat16)
    w2s = (W2 * np.float32(1.0 / np.sqrt(64.0))).astype(jnp.bfloat16)
    w3s = (W3 * np.float32(1.0 / np.sqrt(64.0))).astype(jnp.bfloat16)
    w4s = (W4 * np.float32(1.0 / np.sqrt(64.0))).astype(jnp.bfloat16)
    out_scale = np.float32(1.0 / (np.sqrt(CH) * AVG_NEIGH))
    w_stack = jnp.stack([W_lin0, W_lin1, W_lin1, W_lin1], axis=0) * out_scale

    x = _linear_up(node_feats, w_up_s)
    wv = _edge_weights(ef.reshape(_EPAD // 2, 16), ea.reshape(_EPAD // 2, 8),
                       w1s, w2s, w3s, w4s)
    msg2d = _sc_message(snd, rcv, x, wv.reshape(4 * _EPAD // 2, CH))
    msg = msg2d.reshape(4, N_NODES, CH)
    m = _out_linear(msg, w_stack)
    m1 = jnp.stack([m[1], m[2], m[3]], axis=-1).reshape(N_NODES, 3 * CH)
    return jnp.concatenate([m[0], m1], axis=1)


# speedup vs baseline: 26.4808x; 1.0062x over previous
import functools

import jax
import jax.numpy as jnp
import numpy as np
from jax import lax
from jax.experimental import pallas as pl
from jax.experimental.pallas import tpu as pltpu
from jax.experimental.pallas import tpu_sc as plsc

N_NODES = 10000
N_EDGES = 320000
CH = 128
AVG_NEIGH = 32.0

_NSUB = 16
_B = 64
_NB = 316
_EPT = _B * _NB
_EPAD = _NSUB * _EPT
_ROWS = 624
_TAIL = N_NODES - _NSUB * _ROWS


def _linup_body(nf_ref, w_ref, o_ref):
    o_ref[...] = jnp.dot(nf_ref[...], w_ref[...],
                         preferred_element_type=jnp.float32)


def _linear_up(node_feats, w_up_s):
    bn = 2000
    return pl.pallas_call(
        _linup_body,
        grid=(N_NODES // bn,),
        in_specs=[
            pl.BlockSpec((bn, CH), lambda i: (i, 0)),
            pl.BlockSpec((CH, CH), lambda i: (0, 0)),
        ],
        out_specs=pl.BlockSpec((bn, CH), lambda i: (i, 0)),
        out_shape=jax.ShapeDtypeStruct((N_NODES, CH), jnp.float32),
    )(node_feats, w_up_s)


def _mlp_tc(x, w1_ref, w2_ref, w3_ref, w4_ref):
    h = jax.nn.silu(jnp.dot(x.astype(jnp.bfloat16), w1_ref[...],
                            preferred_element_type=jnp.float32))
    h = jax.nn.silu(jnp.dot(h.astype(jnp.bfloat16), w2_ref[...],
                            preferred_element_type=jnp.float32))
    h = jax.nn.silu(jnp.dot(h.astype(jnp.bfloat16), w3_ref[...],
                            preferred_element_type=jnp.float32))
    return jnp.dot(h.astype(jnp.bfloat16), w4_ref[...],
                   preferred_element_type=jnp.float32)


def _rne_bf16_bits(a):
    u = jax.lax.bitcast_convert_type(a, jnp.int32)
    return jax.lax.shift_right_logical(
        u + jnp.int32(0x7FFF) + (jax.lax.shift_right_logical(u, 16)
                                 & jnp.int32(1)), 16)


def _edgew_body(sel, ef_ref, ea_ref, w1_ref, w2_ref, w3_ref, w4_ref,
                wv_ref):
    efp = ef_ref[...]
    eap = ea_ref[...]
    tpe = _mlp_tc(efp[:, :8], w1_ref, w2_ref, w3_ref, w4_ref)
    tpo = _mlp_tc(efp[:, 8:], w1_ref, w2_ref, w3_ref, w4_ref)
    w0e, w1e = tpe[:, :CH], tpe[:, CH:]
    w0o, w1o = tpo[:, :CH], tpo[:, CH:]

    def pack(a, b):
        return _rne_bf16_bits(a) | (_rne_bf16_bits(b) << 16)

    if sel == 0:
        wv_ref[0] = pack(w0e * eap[:, 0:1], w0o * eap[:, 4:5])
        wv_ref[1] = pack(w1e * eap[:, 1:2], w1o * eap[:, 5:6])
    else:
        wv_ref[0] = pack(w1e * eap[:, 2:3], w1o * eap[:, 6:7])
        wv_ref[1] = pack(w1e * eap[:, 3:4], w1o * eap[:, 7:8])


def _edge_weights(sel, efp, eap, w1s, w2s, w3s, w4s):
    be2 = 2048
    return pl.pallas_call(
        functools.partial(_edgew_body, sel),
        grid=(_EPAD // 2 // be2,),
        in_specs=[
            pl.BlockSpec((be2, 16), lambda i: (i, 0)),
            pl.BlockSpec((be2, 8), lambda i: (i, 0)),
            pl.BlockSpec((8, 64), lambda i: (0, 0)),
            pl.BlockSpec((64, 64), lambda i: (0, 0)),
            pl.BlockSpec((64, 64), lambda i: (0, 0)),
            pl.BlockSpec((64, 2 * CH), lambda i: (0, 0)),
        ],
        out_specs=pl.BlockSpec((2, be2, CH), lambda i: (0, i, 0)),
        out_shape=jax.ShapeDtypeStruct((2, _EPAD // 2, CH), jnp.int32),
    )(efp, eap, w1s, w2s, w3s, w4s)


def _sc_body(snd_hbm, rcv_hbm, x_hbm, wv_hbm, out_hbm,
             snd0, snd1, rcv0, rcv1, ridx0, ridx1,
             xs0, xs1, wv0, wv1, prod0, prod1, acc_sh,
             is0, is1, ir0, ir1, gs0, gs1, ws0, ws1, ss0, ss1):
    cid = lax.axis_index("c")
    sid = lax.axis_index("s")
    ebase = sid * _EPT

    snd_b = (snd0, snd1)
    rcv_b = (rcv0, rcv1)
    ridx_b = (ridx0, ridx1)
    xs_b = (xs0, xs1)
    wv_b = (wv0, wv1)
    prod_b = (prod0, prod1)
    is_b = (is0, is1)
    ir_b = (ir0, ir1)
    gs_b = (gs0, gs1)
    ws_b = (ws0, ws1)
    ss_b = (ss0, ss1)

    for r in range(1):
        chunk = cid
        wvbase = chunk * (_EPAD // 2) + sid * (_EPT // 2)

        def zrow(i, carry):
            for k in range(CH // 16):
                prod0[i, pl.ds(k * 16, 16)] = jnp.zeros((16,), jnp.float32)
            return carry
        lax.fori_loop(0, _B, zrow, 0)
        for z in range(9):
            pltpu.sync_copy(prod0,
                            acc_sh.at[pl.ds(sid * _ROWS + z * _B, _B)])
        pltpu.sync_copy(prod0.at[pl.ds(0, 48)],
                        acc_sh.at[pl.ds(sid * _ROWS + 9 * _B, 48)])

        @pl.when(sid == 0)
        def _zero_tail():
            pltpu.sync_copy(prod0.at[pl.ds(0, _TAIL)],
                            acc_sh.at[pl.ds(_NSUB * _ROWS, _TAIL)])
        plsc.subcore_barrier()

        def idx_start(i, b):
            eb = ebase + i * _B
            pltpu.async_copy(snd_hbm.at[pl.ds(eb, _B)], snd_b[b], is_b[b])
            pltpu.async_copy(rcv_hbm.at[pl.ds(eb, _B)], rcv_b[b], ir_b[b])

        def idx_wait(i, b):
            eb = ebase + i * _B
            pltpu.make_async_copy(snd_hbm.at[pl.ds(eb, _B)], snd_b[b],
                                  is_b[b]).wait()
            pltpu.make_async_copy(rcv_hbm.at[pl.ds(eb, _B)], rcv_b[b],
                                  ir_b[b]).wait()

        def fetch_start(i, b):
            pltpu.async_copy(x_hbm.at[snd_b[b]], xs_b[b], gs_b[b])
            pltpu.async_copy(wv_hbm.at[pl.ds(wvbase + i * (_B // 2),
                                             _B // 2)],
                             wv_b[b], ws_b[b])

        def fetch_wait(i, b):
            pltpu.make_async_copy(x_hbm.at[snd_b[b]], xs_b[b],
                                  gs_b[b]).wait()
            pltpu.make_async_copy(wv_hbm.at[pl.ds(wvbase + i * (_B // 2),
                                                  _B // 2)],
                                  wv_b[b], ws_b[b]).wait()

        def scatter_wait(b):
            pltpu.make_async_copy(prod_b[b], acc_sh.at[ridx_b[b]],
                                  ss_b[b]).wait()

        idx_start(0, 0)
        idx_start(1, 1)
        idx_wait(0, 0)
        fetch_start(0, 0)

        def half(i, b):
            q = 1 - b
            xs_p, wv_p, prod_p = xs_b[b], wv_b[b], prod_b[b]

            @pl.when(i >= 2)
            def _():
                scatter_wait(b)
            fetch_wait(i, b)

            for t in range(_B // 16):
                ridx_b[b][pl.ds(t * 16, 16)] = rcv_b[b][pl.ds(t * 16, 16)]

            @pl.when(i + 2 < _NB)
            def _():
                idx_start(i + 2, b)

            @pl.when(i + 1 < _NB)
            def _():
                idx_wait(i + 1, q)
                fetch_start(i + 1, q)

            @plsc.parallel_loop(0, _B // 2, 1, unroll=4)
            def _mul(m):
                e0 = 2 * m
                for t in range(CH // 16):
                    wvv = wv_p[m, pl.ds(t * 16, 16)]
                    wa = lax.bitcast_convert_type(wvv << 16, jnp.float32)
                    wb = lax.bitcast_convert_type(
                        wvv & jnp.int32(-65536), jnp.float32)
                    prod_p[e0, pl.ds(t * 16, 16)] = (
                        xs_p[e0, pl.ds(t * 16, 16)] * wa)
                    prod_p[e0 + 1, pl.ds(t * 16, 16)] = (
                        xs_p[e0 + 1, pl.ds(t * 16, 16)] * wb)

            pltpu.async_copy(prod_p, acc_sh.at[ridx_b[b]], ss_b[b],
                             add=True)

        def pair(j, carry):
            half(2 * j, 0)
            half(2 * j + 1, 1)
            return carry
        lax.fori_loop(0, _NB // 2, pair, 0)

        scatter_wait(0)
        scatter_wait(1)
        plsc.subcore_barrier()

        pltpu.sync_copy(acc_sh.at[pl.ds(sid * _ROWS, _ROWS)],
                        out_hbm.at[pl.ds(chunk * N_NODES + sid * _ROWS,
                                         _ROWS)])

        @pl.when(sid == 0)
        def _dump_tail():
            pltpu.sync_copy(
                acc_sh.at[pl.ds(_NSUB * _ROWS, _TAIL)],
                out_hbm.at[pl.ds(chunk * N_NODES + _NSUB * _ROWS, _TAIL)])


def _sc_message(snd, rcv, x, wv2d):
    mesh = plsc.VectorSubcoreMesh(core_axis_name="c", subcore_axis_name="s")
    dma = pltpu.SemaphoreType.DMA
    k = functools.partial(
        pl.kernel,
        mesh=mesh,
        out_type=jax.ShapeDtypeStruct((2 * N_NODES, CH), jnp.float32),
        scratch_types=[
            pltpu.VMEM((_B,), jnp.int32),
            pltpu.VMEM((_B,), jnp.int32),
            pltpu.VMEM((_B,), jnp.int32),
            pltpu.VMEM((_B,), jnp.int32),
            pltpu.VMEM((_B,), jnp.int32),
            pltpu.VMEM((_B,), jnp.int32),
            pltpu.VMEM((_B, CH), jnp.float32),
            pltpu.VMEM((_B, CH), jnp.float32),
            pltpu.VMEM((_B // 2, CH), jnp.int32),
            pltpu.VMEM((_B // 2, CH), jnp.int32),
            pltpu.VMEM((_B, CH), jnp.float32),
            pltpu.VMEM((_B, CH), jnp.float32),
            pltpu.VMEM_SHARED((N_NODES, CH), jnp.float32),
            dma, dma, dma, dma, dma, dma, dma, dma, dma, dma,
        ],
    )(_sc_body)
    return k(snd, rcv, x, wv2d)


def _outlin_body(m_ref, w_ref, o_ref):
    o_ref[0] = jnp.dot(m_ref[0], w_ref[0],
                       preferred_element_type=jnp.float32)


def _out_linear(msg, w_stack):
    bn = 2000
    return pl.pallas_call(
        _outlin_body,
        grid=(4, N_NODES // bn),
        in_specs=[
            pl.BlockSpec((1, bn, CH), lambda c, i: (c, i, 0)),
            pl.BlockSpec((1, CH, CH), lambda c, i: (c, 0, 0)),
        ],
        out_specs=pl.BlockSpec((1, bn, CH), lambda c, i: (c, i, 0)),
        out_shape=jax.ShapeDtypeStruct((4, N_NODES, CH), jnp.float32),
    )(msg, w_stack)


def kernel(node_attrs, node_feats, edge_attrs, edge_feats, edge_index,
           W_up, W1, W2, W3, W4, W_lin0, W_lin1):
    del node_attrs
    pad = _EPAD - N_EDGES
    snd = jnp.pad(edge_index[0], (0, pad))
    rcv = jnp.pad(edge_index[1], (0, pad))
    ef = jnp.pad(edge_feats, ((0, pad), (0, 0)))
    ea = jnp.pad(edge_attrs, ((0, pad), (0, 0)))
    w_up_s = W_up * np.float32(1.0 / np.sqrt(CH))
    w1s = (W1 * np.float32(1.0 / np.sqrt(8.0))).astype(jnp.bfloat16)
    w2s = (W2 * np.float32(1.0 / np.sqrt(64.0))).astype(jnp.bfloat16)
    w3s = (W3 * np.float32(1.0 / np.sqrt(64.0))).astype(jnp.bfloat16)
    w4s = (W4 * np.float32(1.0 / np.sqrt(64.0))).astype(jnp.bfloat16)
    out_scale = np.float32(1.0 / (np.sqrt(CH) * AVG_NEIGH))
    w_stack = jnp.stack([W_lin0, W_lin1, W_lin1, W_lin1], axis=0) * out_scale

    x = _linear_up(node_feats, w_up_s)
    efp = ef.reshape(_EPAD // 2, 16)
    eap = ea.reshape(_EPAD // 2, 8)
    wva = _edge_weights(0, efp, eap, w1s, w2s, w3s, w4s)
    wvb = _edge_weights(1, efp, eap, w1s, w2s, w3s, w4s)
    ma = _sc_message(snd, rcv, x, wva.reshape(2 * _EPAD // 2, CH))
    mb = _sc_message(snd, rcv, x, wvb.reshape(2 * _EPAD // 2, CH))
    msg = jnp.concatenate([ma, mb], axis=0).reshape(4, N_NODES, CH)
    m = _out_linear(msg, w_stack)
    m1 = jnp.stack([m[1], m[2], m[3]], axis=-1).reshape(N_NODES, 3 * CH)
    return jnp.concatenate([m[0], m1], axis=1)
